# Initial kernel scaffold; baseline (speedup 1.0000x reference)
import functools

import jax
import jax.numpy as jnp
from jax import lax
from jax.experimental import pallas as pl
from jax.experimental.pallas import tpu as pltpu
from jax.experimental.pallas import tpu_sc as plsc

N = 10000
E = 320000
NODE_DIM = 128
EDGE_DIM = 16
GLOBAL_DIM = 64
HIDDEN = 64
G = 64

NC = 2
NS = 16
N_PAD = 10240
ROWS_PER_SUB = N_PAD // NS
EDGES_PER_W = E // (NC * NS)
CHUNK = 128
FULL_CHUNKS = EDGES_PER_W // CHUNK
TAIL = EDGES_PER_W - FULL_CHUNKS * CHUNK


def _sc_segment_sum(edges, recv):
    mesh = plsc.VectorSubcoreMesh(core_axis_name="c", subcore_axis_name="s")

    @functools.partial(
        pl.kernel,
        out_type=jax.ShapeDtypeStruct((NC, N_PAD, EDGE_DIM), jnp.float32),
        mesh=mesh,
        scratch_types=[
            pltpu.VMEM((CHUNK,), jnp.int32),
            pltpu.VMEM((CHUNK, EDGE_DIM), jnp.float32),
            pltpu.VMEM((TAIL,), jnp.int32),
            pltpu.VMEM((TAIL, EDGE_DIM), jnp.float32),
            pltpu.VMEM((CHUNK, EDGE_DIM), jnp.float32),
            pltpu.VMEM_SHARED((N_PAD, EDGE_DIM), jnp.float32),
        ],
    )
    def kern(edges_hbm, recv_hbm, out_hbm, idx_v, ed_v, tidx_v, ted_v,
             zero_v, acc_sh):
        c = lax.axis_index("c")
        s = lax.axis_index("s")

        @pl.loop(0, CHUNK)
        def _(i):
            zero_v[i, :] = jnp.zeros((EDGE_DIM,), jnp.float32)

        @pl.loop(0, ROWS_PER_SUB // CHUNK)
        def _(j):
            pltpu.sync_copy(zero_v,
                            acc_sh.at[pl.ds(s * ROWS_PER_SUB + j * CHUNK,
                                            CHUNK)])

        plsc.subcore_barrier()

        base = (c * NS + s) * EDGES_PER_W

        @pl.loop(0, FULL_CHUNKS)
        def _(j):
            off = base + j * CHUNK
            pltpu.sync_copy(recv_hbm.at[pl.ds(off, CHUNK)], idx_v)
            pltpu.sync_copy(edges_hbm.at[pl.ds(off, CHUNK)], ed_v)
            pltpu.sync_copy(ed_v, acc_sh.at[idx_v], add=True)

        toff = base + FULL_CHUNKS * CHUNK
        pltpu.sync_copy(recv_hbm.at[pl.ds(toff, TAIL)], tidx_v)
        pltpu.sync_copy(edges_hbm.at[pl.ds(toff, TAIL)], ted_v)
        pltpu.sync_copy(ted_v, acc_sh.at[tidx_v], add=True)

        plsc.subcore_barrier()

        pltpu.sync_copy(acc_sh.at[pl.ds(s * ROWS_PER_SUB, ROWS_PER_SUB)],
                        out_hbm.at[c, pl.ds(s * ROWS_PER_SUB, ROWS_PER_SUB)])

    return kern(edges, recv)


BLK = 1000


def _tc_mlp_kernel(nodes_ref, p_ref, batch_ref, gg_ref, w1_ref, b1_ref,
                   w2_ref, b2_ref, out_ref):
    f32 = jnp.float32
    nodes = nodes_ref[...]
    p = p_ref[...]
    agg = p[0] + p[1]
    bt = batch_ref[0, 0, :]
    gh = jnp.dot(gg_ref[...], w1_ref[NODE_DIM + EDGE_DIM:, :],
                 preferred_element_type=f32)
    onehot = (bt[:, None] == lax.broadcasted_iota(jnp.int32, (BLK, G), 1)
              ).astype(f32)
    acc = (jnp.dot(nodes, w1_ref[:NODE_DIM, :], preferred_element_type=f32)
           + jnp.dot(agg, w1_ref[NODE_DIM:NODE_DIM + EDGE_DIM, :],
                     preferred_element_type=f32)
           + jnp.dot(onehot, gh, preferred_element_type=f32)
           + b1_ref[...])
    h = jnp.maximum(acc, 0.0)
    out_ref[...] = jnp.dot(h, w2_ref[...], preferred_element_type=f32) \
        + b2_ref[...]


def kernel(nodes, edges, edge_index, graph_globals, batch, W1, b1, W2, b2):
    recv = edge_index[1]
    partials = _sc_segment_sum(edges, recv)

    batch3 = batch.reshape(N // BLK, 1, BLK)
    grid = (N // BLK,)
    out = pl.pallas_call(
        _tc_mlp_kernel,
        grid=grid,
        in_specs=[
            pl.BlockSpec((BLK, NODE_DIM), lambda i: (i, 0)),
            pl.BlockSpec((NC, BLK, EDGE_DIM), lambda i: (0, i, 0)),
            pl.BlockSpec((1, 1, BLK), lambda i: (i, 0, 0)),
            pl.BlockSpec((G, GLOBAL_DIM), lambda i: (0, 0)),
            pl.BlockSpec((NODE_DIM + EDGE_DIM + GLOBAL_DIM, HIDDEN),
                         lambda i: (0, 0)),
            pl.BlockSpec((1, HIDDEN), lambda i: (0, 0)),
            pl.BlockSpec((HIDDEN, NODE_DIM), lambda i: (0, 0)),
            pl.BlockSpec((1, NODE_DIM), lambda i: (0, 0)),
        ],
        out_specs=pl.BlockSpec((BLK, NODE_DIM), lambda i: (i, 0)),
        out_shape=jax.ShapeDtypeStruct((N, NODE_DIM), jnp.float32),
    )(nodes, partials, batch3, graph_globals, W1, b1.reshape(1, HIDDEN),
      W2, b2.reshape(1, NODE_DIM))
    return out


# speedup vs baseline: 3.7699x; 3.7699x over previous
import functools

import jax
import jax.numpy as jnp
from jax import lax
from jax.experimental import pallas as pl
from jax.experimental.pallas import tpu as pltpu
from jax.experimental.pallas import tpu_sc as plsc

N = 10000
E = 320000
NODE_DIM = 128
EDGE_DIM = 16
GLOBAL_DIM = 64
HIDDEN = 64
G = 64

NC = 2
NS = 16
NW = NC * NS
N_PAD = 10240
ROWS_PER_SUB = N_PAD // NS
CHUNK = 128
NUM_CHUNKS = E // CHUNK
BASE_CHUNKS = NUM_CHUNKS // NW
EXTRA = NUM_CHUNKS - BASE_CHUNKS * NW


def _sc_segment_sum(edges, recv2):
    mesh = plsc.VectorSubcoreMesh(core_axis_name="c", subcore_axis_name="s")

    @functools.partial(
        pl.kernel,
        out_type=jax.ShapeDtypeStruct((NC, N_PAD, EDGE_DIM), jnp.float32),
        mesh=mesh,
        compiler_params=pltpu.CompilerParams(use_tc_tiling_on_sc=False),
        scratch_types=[
            pltpu.VMEM((1, CHUNK), jnp.int32),
            pltpu.VMEM((CHUNK, EDGE_DIM), jnp.float32),
            pltpu.VMEM((CHUNK, EDGE_DIM), jnp.float32),
            pltpu.VMEM_SHARED((N_PAD, EDGE_DIM), jnp.float32),
        ],
    )
    def kern(edges_hbm, recv_hbm, out_hbm, idx_v, ed_v, zero_v, acc_sh):
        c = lax.axis_index("c")
        s = lax.axis_index("s")
        w = c * NS + s

        @pl.loop(0, CHUNK)
        def _(i):
            zero_v[i, :] = jnp.zeros((EDGE_DIM,), jnp.float32)

        @pl.loop(0, ROWS_PER_SUB // CHUNK)
        def _(j):
            pltpu.sync_copy(zero_v,
                            acc_sh.at[pl.ds(s * ROWS_PER_SUB + j * CHUNK,
                                            CHUNK)])

        plsc.subcore_barrier()

        lo = w * BASE_CHUNKS + jnp.minimum(w, EXTRA)
        cnt = BASE_CHUNKS + jnp.where(w < EXTRA, 1, 0)

        @pl.loop(0, cnt)
        def _(j):
            k = lo + j
            pltpu.sync_copy(recv_hbm.at[pl.ds(k, 1)], idx_v)
            pltpu.sync_copy(edges_hbm.at[pl.ds(k * CHUNK, CHUNK)], ed_v)
            pltpu.sync_copy(ed_v, acc_sh.at[idx_v.at[0]], add=True)

        plsc.subcore_barrier()

        pltpu.sync_copy(acc_sh.at[pl.ds(s * ROWS_PER_SUB, ROWS_PER_SUB)],
                        out_hbm.at[c, pl.ds(s * ROWS_PER_SUB, ROWS_PER_SUB)])

    return kern(edges, recv2)


BLK = 1000


def _tc_mlp_kernel(nodes_ref, p_ref, batch_ref, gg_ref, w1_ref, b1_ref,
                   w2_ref, b2_ref, out_ref):
    f32 = jnp.float32
    nodes = nodes_ref[...]
    p = p_ref[...]
    agg = p[0] + p[1]
    bt = batch_ref[0, 0, :]
    gh = jnp.dot(gg_ref[...], w1_ref[NODE_DIM + EDGE_DIM:, :],
                 preferred_element_type=f32)
    onehot = (bt[:, None] == lax.broadcasted_iota(jnp.int32, (BLK, G), 1)
              ).astype(f32)
    acc = (jnp.dot(nodes, w1_ref[:NODE_DIM, :], preferred_element_type=f32)
           + jnp.dot(agg, w1_ref[NODE_DIM:NODE_DIM + EDGE_DIM, :],
                     preferred_element_type=f32)
           + jnp.dot(onehot, gh, preferred_element_type=f32)
           + b1_ref[...])
    h = jnp.maximum(acc, 0.0)
    out_ref[...] = jnp.dot(h, w2_ref[...], preferred_element_type=f32) \
        + b2_ref[...]


def kernel(nodes, edges, edge_index, graph_globals, batch, W1, b1, W2, b2):
    recv2 = edge_index[1].reshape(NUM_CHUNKS, CHUNK)
    partials = _sc_segment_sum(edges, recv2)

    batch3 = batch.reshape(N // BLK, 1, BLK)
    grid = (N // BLK,)
    out = pl.pallas_call(
        _tc_mlp_kernel,
        grid=grid,
        in_specs=[
            pl.BlockSpec((BLK, NODE_DIM), lambda i: (i, 0)),
            pl.BlockSpec((NC, BLK, EDGE_DIM), lambda i: (0, i, 0)),
            pl.BlockSpec((1, 1, BLK), lambda i: (i, 0, 0)),
            pl.BlockSpec((G, GLOBAL_DIM), lambda i: (0, 0)),
            pl.BlockSpec((NODE_DIM + EDGE_DIM + GLOBAL_DIM, HIDDEN),
                         lambda i: (0, 0)),
            pl.BlockSpec((1, HIDDEN), lambda i: (0, 0)),
            pl.BlockSpec((HIDDEN, NODE_DIM), lambda i: (0, 0)),
            pl.BlockSpec((1, NODE_DIM), lambda i: (0, 0)),
        ],
        out_specs=pl.BlockSpec((BLK, NODE_DIM), lambda i: (i, 0)),
        out_shape=jax.ShapeDtypeStruct((N, NODE_DIM), jnp.float32),
    )(nodes, partials, batch3, graph_globals, W1, b1.reshape(1, HIDDEN),
      W2, b2.reshape(1, NODE_DIM))
    return out


# speedup vs baseline: 4.9815x; 1.3214x over previous
import functools

import jax
import jax.numpy as jnp
from jax import lax
from jax.experimental import pallas as pl
from jax.experimental.pallas import tpu as pltpu
from jax.experimental.pallas import tpu_sc as plsc

N = 10000
E = 320000
NODE_DIM = 128
EDGE_DIM = 16
GLOBAL_DIM = 64
HIDDEN = 64
G = 64

NC = 2
NS = 16
NW = NC * NS
N_PAD = 10240
ROWS_PER_SUB = N_PAD // NS
CHUNK = 128
NUM_CHUNKS = E // CHUNK
BASE_CHUNKS = NUM_CHUNKS // NW
EXTRA = NUM_CHUNKS - BASE_CHUNKS * NW


def _sc_segment_sum(edges, edge_index):
    mesh = plsc.VectorSubcoreMesh(core_axis_name="c", subcore_axis_name="s")

    @functools.partial(
        pl.kernel,
        out_type=jax.ShapeDtypeStruct((NC, N_PAD, EDGE_DIM), jnp.float32),
        mesh=mesh,
        compiler_params=pltpu.CompilerParams(use_tc_tiling_on_sc=False),
        scratch_types=[
            pltpu.VMEM((2, CHUNK), jnp.int32),
            pltpu.VMEM((CHUNK, EDGE_DIM), jnp.float32),
            pltpu.VMEM((CHUNK, EDGE_DIM), jnp.float32),
            pltpu.VMEM((CHUNK, EDGE_DIM), jnp.float32),
            pltpu.VMEM_SHARED((N_PAD, EDGE_DIM), jnp.float32),
            pltpu.SemaphoreType.DMA((2,)),
            pltpu.SemaphoreType.DMA((2,)),
        ],
    )
    def kern(edges_hbm, ei_hbm, out_hbm, idx_v, ed0_v, ed1_v, zero_v,
             acc_sh, fsem, ssem):
        c = lax.axis_index("c")
        s = lax.axis_index("s")
        w = c * NS + s
        ed_bufs = (ed0_v, ed1_v)

        def fetch(k, b):
            pltpu.async_copy(
                ei_hbm.at[pl.ds(1, 1), pl.ds(k * CHUNK, CHUNK)],
                idx_v.at[pl.ds(b, 1)], fsem.at[b])
            pltpu.async_copy(
                edges_hbm.at[pl.ds(k * CHUNK, CHUNK)],
                ed_bufs[b], fsem.at[b])

        def wait_fetch(k, b):
            pltpu.make_async_copy(
                ei_hbm.at[pl.ds(1, 1), pl.ds(k * CHUNK, CHUNK)],
                idx_v.at[pl.ds(b, 1)], fsem.at[b]).wait()
            pltpu.make_async_copy(
                edges_hbm.at[pl.ds(k * CHUNK, CHUNK)],
                ed_bufs[b], fsem.at[b]).wait()

        def scatter_start(b):
            pltpu.async_copy(
                ed_bufs[b], acc_sh.at[idx_v.at[b]], ssem.at[b], add=True)

        def scatter_wait(b):
            pltpu.make_async_copy(
                ed_bufs[b], acc_sh.at[idx_v.at[b]], ssem.at[b]).wait()

        @pl.loop(0, CHUNK)
        def _(i):
            zero_v[i, :] = jnp.zeros((EDGE_DIM,), jnp.float32)

        @pl.loop(0, ROWS_PER_SUB // CHUNK)
        def _(j):
            pltpu.sync_copy(zero_v,
                            acc_sh.at[pl.ds(s * ROWS_PER_SUB + j * CHUNK,
                                            CHUNK)])

        plsc.subcore_barrier()

        lo = w * BASE_CHUNKS + jnp.minimum(w, EXTRA)
        cnt = BASE_CHUNKS + jnp.where(w < EXTRA, 1, 0)
        hi = lo + cnt

        fetch(lo, 0)
        fetch(lo + 1, 1)

        @pl.loop(0, BASE_CHUNKS // 2)
        def _(jj):
            k0 = lo + 2 * jj

            wait_fetch(k0, 0)
            scatter_start(0)
            wait_fetch(k0 + 1, 1)
            scatter_start(1)

            scatter_wait(0)

            @pl.when(k0 + 2 < hi)
            def _():
                fetch(k0 + 2, 0)

            scatter_wait(1)

            @pl.when(k0 + 3 < hi)
            def _():
                fetch(k0 + 3, 1)

        @pl.when(cnt > BASE_CHUNKS)
        def _():
            wait_fetch(hi - 1, 0)
            scatter_start(0)
            scatter_wait(0)

        plsc.subcore_barrier()

        pltpu.sync_copy(acc_sh.at[pl.ds(s * ROWS_PER_SUB, ROWS_PER_SUB)],
                        out_hbm.at[c, pl.ds(s * ROWS_PER_SUB, ROWS_PER_SUB)])

    return kern(edges, edge_index)


BLK = 1000


def _tc_mlp_kernel(nodes_ref, p_ref, batch_ref, gg_ref, w1_ref, b1_ref,
                   w2_ref, b2_ref, out_ref):
    f32 = jnp.float32
    nodes = nodes_ref[...]
    p = p_ref[...]
    agg = p[0] + p[1]
    bt = batch_ref[0, 0, :]
    gh = jnp.dot(gg_ref[...], w1_ref[NODE_DIM + EDGE_DIM:, :],
                 preferred_element_type=f32)
    onehot = (bt[:, None] == lax.broadcasted_iota(jnp.int32, (BLK, G), 1)
              ).astype(f32)
    acc = (jnp.dot(nodes, w1_ref[:NODE_DIM, :], preferred_element_type=f32)
           + jnp.dot(agg, w1_ref[NODE_DIM:NODE_DIM + EDGE_DIM, :],
                     preferred_element_type=f32)
           + jnp.dot(onehot, gh, preferred_element_type=f32)
           + b1_ref[...])
    h = jnp.maximum(acc, 0.0)
    out_ref[...] = jnp.dot(h, w2_ref[...], preferred_element_type=f32) \
        + b2_ref[...]


def kernel(nodes, edges, edge_index, graph_globals, batch, W1, b1, W2, b2):
    partials = _sc_segment_sum(edges, edge_index)

    batch3 = batch.reshape(N // BLK, 1, BLK)
    grid = (N // BLK,)
    out = pl.pallas_call(
        _tc_mlp_kernel,
        grid=grid,
        in_specs=[
            pl.BlockSpec((BLK, NODE_DIM), lambda i: (i, 0)),
            pl.BlockSpec((NC, BLK, EDGE_DIM), lambda i: (0, i, 0)),
            pl.BlockSpec((1, 1, BLK), lambda i: (i, 0, 0)),
            pl.BlockSpec((G, GLOBAL_DIM), lambda i: (0, 0)),
            pl.BlockSpec((NODE_DIM + EDGE_DIM + GLOBAL_DIM, HIDDEN),
                         lambda i: (0, 0)),
            pl.BlockSpec((1, HIDDEN), lambda i: (0, 0)),
            pl.BlockSpec((HIDDEN, NODE_DIM), lambda i: (0, 0)),
            pl.BlockSpec((1, NODE_DIM), lambda i: (0, 0)),
        ],
        out_specs=pl.BlockSpec((BLK, NODE_DIM), lambda i: (i, 0)),
        out_shape=jax.ShapeDtypeStruct((N, NODE_DIM), jnp.float32),
    )(nodes, partials, batch3, graph_globals, W1, b1.reshape(1, HIDDEN),
      W2, b2.reshape(1, NODE_DIM))
    return out


# speedup vs baseline: 4.9833x; 1.0004x over previous
import functools

import jax
import jax.numpy as jnp
from jax import lax
from jax.experimental import pallas as pl
from jax.experimental.pallas import tpu as pltpu
from jax.experimental.pallas import tpu_sc as plsc

N = 10000
E = 320000
NODE_DIM = 128
EDGE_DIM = 16
GLOBAL_DIM = 64
HIDDEN = 64
G = 64

NC = 2
NS = 16
NW = NC * NS
N_PAD = 10240
ROWS_PER_SUB = N_PAD // NS
CHUNK = 128
NUM_CHUNKS = E // CHUNK
BASE_CHUNKS = NUM_CHUNKS // NW
EXTRA = NUM_CHUNKS - BASE_CHUNKS * NW


EXB = 32768
CHUNKS_PAD = 2560


def _extract_recv_kernel(ei_ref, out_ref):
    row = ei_ref[1, :]
    out_ref[...] = row.reshape(EXB // CHUNK, CHUNK)


def _extract_recv(edge_index):
    return pl.pallas_call(
        _extract_recv_kernel,
        grid=(CHUNKS_PAD * CHUNK // EXB,),
        in_specs=[pl.BlockSpec((2, EXB), lambda i: (0, i))],
        out_specs=pl.BlockSpec((EXB // CHUNK, CHUNK), lambda i: (i, 0)),
        out_shape=jax.ShapeDtypeStruct((CHUNKS_PAD, CHUNK), jnp.int32),
    )(edge_index)


def _sc_segment_sum(edges, recv2):
    mesh = plsc.VectorSubcoreMesh(core_axis_name="c", subcore_axis_name="s")

    @functools.partial(
        pl.kernel,
        out_type=jax.ShapeDtypeStruct((NC, N_PAD, EDGE_DIM), jnp.float32),
        mesh=mesh,
        compiler_params=pltpu.CompilerParams(use_tc_tiling_on_sc=False),
        scratch_types=[
            pltpu.VMEM((2, CHUNK), jnp.int32),
            pltpu.VMEM((CHUNK, EDGE_DIM), jnp.float32),
            pltpu.VMEM((CHUNK, EDGE_DIM), jnp.float32),
            pltpu.VMEM((CHUNK, EDGE_DIM), jnp.float32),
            pltpu.VMEM_SHARED((N_PAD, EDGE_DIM), jnp.float32),
            pltpu.SemaphoreType.DMA((2,)),
            pltpu.SemaphoreType.DMA((2,)),
        ],
    )
    def kern(edges_hbm, recv_hbm, out_hbm, idx_v, ed0_v, ed1_v, zero_v,
             acc_sh, fsem, ssem):
        c = lax.axis_index("c")
        s = lax.axis_index("s")
        w = c * NS + s
        ed_bufs = (ed0_v, ed1_v)

        def fetch(k, b):
            pltpu.async_copy(
                recv_hbm.at[pl.ds(k, 1)],
                idx_v.at[pl.ds(b, 1)], fsem.at[b])
            pltpu.async_copy(
                edges_hbm.at[pl.ds(k * CHUNK, CHUNK)],
                ed_bufs[b], fsem.at[b])

        def wait_fetch(k, b):
            pltpu.make_async_copy(
                recv_hbm.at[pl.ds(k, 1)],
                idx_v.at[pl.ds(b, 1)], fsem.at[b]).wait()
            pltpu.make_async_copy(
                edges_hbm.at[pl.ds(k * CHUNK, CHUNK)],
                ed_bufs[b], fsem.at[b]).wait()

        def scatter_start(b):
            pltpu.async_copy(
                ed_bufs[b], acc_sh.at[idx_v.at[b]], ssem.at[b], add=True)

        def scatter_wait(b):
            pltpu.make_async_copy(
                ed_bufs[b], acc_sh.at[idx_v.at[b]], ssem.at[b]).wait()

        @pl.loop(0, CHUNK)
        def _(i):
            zero_v[i, :] = jnp.zeros((EDGE_DIM,), jnp.float32)

        @pl.loop(0, ROWS_PER_SUB // CHUNK)
        def _(j):
            pltpu.sync_copy(zero_v,
                            acc_sh.at[pl.ds(s * ROWS_PER_SUB + j * CHUNK,
                                            CHUNK)])

        plsc.subcore_barrier()

        lo = w * BASE_CHUNKS + jnp.minimum(w, EXTRA)
        cnt = BASE_CHUNKS + jnp.where(w < EXTRA, 1, 0)
        hi = lo + cnt

        fetch(lo, 0)
        fetch(lo + 1, 1)

        @pl.loop(0, BASE_CHUNKS // 2)
        def _(jj):
            k0 = lo + 2 * jj

            wait_fetch(k0, 0)
            scatter_start(0)
            wait_fetch(k0 + 1, 1)
            scatter_start(1)

            scatter_wait(0)

            @pl.when(k0 + 2 < hi)
            def _():
                fetch(k0 + 2, 0)

            scatter_wait(1)

            @pl.when(k0 + 3 < hi)
            def _():
                fetch(k0 + 3, 1)

        @pl.when(cnt > BASE_CHUNKS)
        def _():
            wait_fetch(hi - 1, 0)
            scatter_start(0)
            scatter_wait(0)

        plsc.subcore_barrier()

        pltpu.sync_copy(acc_sh.at[pl.ds(s * ROWS_PER_SUB, ROWS_PER_SUB)],
                        out_hbm.at[c, pl.ds(s * ROWS_PER_SUB, ROWS_PER_SUB)])

    return kern(edges, recv2)


BLK = 1000


def _tc_mlp_kernel(nodes_ref, p_ref, batch_ref, gg_ref, w1_ref, b1_ref,
                   w2_ref, b2_ref, out_ref):
    f32 = jnp.float32
    nodes = nodes_ref[...]
    p = p_ref[...]
    agg = p[0] + p[1]
    bt = batch_ref[0, 0, :]
    gh = jnp.dot(gg_ref[...], w1_ref[NODE_DIM + EDGE_DIM:, :],
                 preferred_element_type=f32)
    onehot = (bt[:, None] == lax.broadcasted_iota(jnp.int32, (BLK, G), 1)
              ).astype(f32)
    acc = (jnp.dot(nodes, w1_ref[:NODE_DIM, :], preferred_element_type=f32)
           + jnp.dot(agg, w1_ref[NODE_DIM:NODE_DIM + EDGE_DIM, :],
                     preferred_element_type=f32)
           + jnp.dot(onehot, gh, preferred_element_type=f32)
           + b1_ref[...])
    h = jnp.maximum(acc, 0.0)
    out_ref[...] = jnp.dot(h, w2_ref[...], preferred_element_type=f32) \
        + b2_ref[...]


def kernel(nodes, edges, edge_index, graph_globals, batch, W1, b1, W2, b2):
    recv2 = _extract_recv(edge_index)
    partials = _sc_segment_sum(edges, recv2)

    batch3 = batch.reshape(N // BLK, 1, BLK)
    grid = (N // BLK,)
    out = pl.pallas_call(
        _tc_mlp_kernel,
        grid=grid,
        in_specs=[
            pl.BlockSpec((BLK, NODE_DIM), lambda i: (i, 0)),
            pl.BlockSpec((NC, BLK, EDGE_DIM), lambda i: (0, i, 0)),
            pl.BlockSpec((1, 1, BLK), lambda i: (i, 0, 0)),
            pl.BlockSpec((G, GLOBAL_DIM), lambda i: (0, 0)),
            pl.BlockSpec((NODE_DIM + EDGE_DIM + GLOBAL_DIM, HIDDEN),
                         lambda i: (0, 0)),
            pl.BlockSpec((1, HIDDEN), lambda i: (0, 0)),
            pl.BlockSpec((HIDDEN, NODE_DIM), lambda i: (0, 0)),
            pl.BlockSpec((1, NODE_DIM), lambda i: (0, 0)),
        ],
        out_specs=pl.BlockSpec((BLK, NODE_DIM), lambda i: (i, 0)),
        out_shape=jax.ShapeDtypeStruct((N, NODE_DIM), jnp.float32),
    )(nodes, partials, batch3, graph_globals, W1, b1.reshape(1, HIDDEN),
      W2, b2.reshape(1, NODE_DIM))
    return out


# speedup vs baseline: 5.5601x; 1.1157x over previous
import functools

import jax
import jax.numpy as jnp
from jax import lax
from jax.experimental import pallas as pl
from jax.experimental.pallas import tpu as pltpu
from jax.experimental.pallas import tpu_sc as plsc

N = 10000
E = 320000
NODE_DIM = 128
EDGE_DIM = 16
GLOBAL_DIM = 64
HIDDEN = 64
G = 64

NC = 2
NS = 16
NW = NC * NS
N_PAD = 10240
ROWS_PER_SUB = N_PAD // NS
CHUNK = 128
NUM_CHUNKS = E // CHUNK
BASE_CHUNKS = NUM_CHUNKS // NW
EXTRA = NUM_CHUNKS - BASE_CHUNKS * NW


EXB = 32768
CHUNKS_PAD = 2560


def _extract_recv_kernel(ei_ref, out_ref):
    row = ei_ref[1, :]
    out_ref[...] = row.reshape(EXB // CHUNK, CHUNK)


def _extract_recv(edge_index):
    return pl.pallas_call(
        _extract_recv_kernel,
        grid=(CHUNKS_PAD * CHUNK // EXB,),
        in_specs=[pl.BlockSpec((2, EXB), lambda i: (0, i))],
        out_specs=pl.BlockSpec((EXB // CHUNK, CHUNK), lambda i: (i, 0)),
        out_shape=jax.ShapeDtypeStruct((CHUNKS_PAD, CHUNK), jnp.int32),
    )(edge_index)


SLAB_B = 32768


def _edges_to_slabs_kernel(et_ref, out_ref):
    x = et_ref[...]
    y = x.T
    z = y.reshape(SLAB_B // CHUNK, 8, 16, 16)
    z = z.transpose(0, 2, 1, 3)
    out_ref[...] = z.reshape(SLAB_B // CHUNK * 16, 128)


def _edges_to_slabs(edges):
    return pl.pallas_call(
        _edges_to_slabs_kernel,
        grid=(CHUNKS_PAD * CHUNK // SLAB_B,),
        in_specs=[pl.BlockSpec((EDGE_DIM, SLAB_B), lambda i: (0, i))],
        out_specs=pl.BlockSpec((SLAB_B // CHUNK * 16, 128),
                               lambda i: (i, 0)),
        out_shape=jax.ShapeDtypeStruct((CHUNKS_PAD * 16, 128), jnp.float32),
    )(edges.T)


def _sc_segment_sum(edges_slabs, recv2):
    mesh = plsc.VectorSubcoreMesh(core_axis_name="c", subcore_axis_name="s")

    @functools.partial(
        pl.kernel,
        out_type=jax.ShapeDtypeStruct((NC, N_PAD, EDGE_DIM), jnp.float32),
        mesh=mesh,
        compiler_params=pltpu.CompilerParams(use_tc_tiling_on_sc=False),
        scratch_types=[
            pltpu.VMEM((2, CHUNK), jnp.int32),
            pltpu.VMEM((CHUNK, EDGE_DIM), jnp.float32),
            pltpu.VMEM((CHUNK, EDGE_DIM), jnp.float32),
            pltpu.VMEM((CHUNK, EDGE_DIM), jnp.float32),
            pltpu.VMEM_SHARED((N_PAD, EDGE_DIM), jnp.float32),
            pltpu.SemaphoreType.DMA((2,)),
            pltpu.SemaphoreType.DMA((2,)),
        ],
    )
    def kern(edges_hbm, recv_hbm, out_hbm, idx_v, ed0_v, ed1_v, zero_v,
             acc_sh, fsem, ssem):
        c = lax.axis_index("c")
        s = lax.axis_index("s")
        w = c * NS + s
        ed_bufs = (ed0_v, ed1_v)

        def fetch(k, b):
            pltpu.async_copy(
                recv_hbm.at[pl.ds(k, 1)],
                idx_v.at[pl.ds(b, 1)], fsem.at[b])
            for j in range(8):
                pltpu.async_copy(
                    edges_hbm.at[pl.ds(k * 16, 16), pl.ds(j * 16, 16)],
                    ed_bufs[b].at[pl.ds(j * 16, 16), :], fsem.at[b])

        def wait_fetch(k, b):
            pltpu.make_async_copy(
                recv_hbm.at[pl.ds(k, 1)],
                idx_v.at[pl.ds(b, 1)], fsem.at[b]).wait()
            for j in range(8):
                pltpu.make_async_copy(
                    edges_hbm.at[pl.ds(k * 16, 16), pl.ds(j * 16, 16)],
                    ed_bufs[b].at[pl.ds(j * 16, 16), :], fsem.at[b]).wait()

        def scatter_start(b):
            pltpu.async_copy(
                ed_bufs[b], acc_sh.at[idx_v.at[b]], ssem.at[b], add=True)

        def scatter_wait(b):
            pltpu.make_async_copy(
                ed_bufs[b], acc_sh.at[idx_v.at[b]], ssem.at[b]).wait()

        @pl.loop(0, CHUNK)
        def _(i):
            zero_v[i, :] = jnp.zeros((EDGE_DIM,), jnp.float32)

        @pl.loop(0, ROWS_PER_SUB // CHUNK)
        def _(j):
            pltpu.sync_copy(zero_v,
                            acc_sh.at[pl.ds(s * ROWS_PER_SUB + j * CHUNK,
                                            CHUNK)])

        plsc.subcore_barrier()

        lo = w * BASE_CHUNKS + jnp.minimum(w, EXTRA)
        cnt = BASE_CHUNKS + jnp.where(w < EXTRA, 1, 0)
        hi = lo + cnt

        fetch(lo, 0)
        fetch(lo + 1, 1)

        @pl.loop(0, BASE_CHUNKS // 2)
        def _(jj):
            k0 = lo + 2 * jj

            wait_fetch(k0, 0)
            scatter_start(0)
            wait_fetch(k0 + 1, 1)
            scatter_start(1)

            scatter_wait(0)

            @pl.when(k0 + 2 < hi)
            def _():
                fetch(k0 + 2, 0)

            scatter_wait(1)

            @pl.when(k0 + 3 < hi)
            def _():
                fetch(k0 + 3, 1)

        @pl.when(cnt > BASE_CHUNKS)
        def _():
            wait_fetch(hi - 1, 0)
            scatter_start(0)
            scatter_wait(0)

        plsc.subcore_barrier()

        pltpu.sync_copy(acc_sh.at[pl.ds(s * ROWS_PER_SUB, ROWS_PER_SUB)],
                        out_hbm.at[c, pl.ds(s * ROWS_PER_SUB, ROWS_PER_SUB)])

    return kern(edges_slabs, recv2)


BLK = 1000


def _tc_mlp_kernel(nodes_ref, p_ref, batch_ref, gg_ref, w1_ref, b1_ref,
                   w2_ref, b2_ref, out_ref):
    f32 = jnp.float32
    nodes = nodes_ref[...]
    p = p_ref[...]
    agg = p[0] + p[1]
    bt = batch_ref[0, 0, :]
    gh = jnp.dot(gg_ref[...], w1_ref[NODE_DIM + EDGE_DIM:, :],
                 preferred_element_type=f32)
    onehot = (bt[:, None] == lax.broadcasted_iota(jnp.int32, (BLK, G), 1)
              ).astype(f32)
    acc = (jnp.dot(nodes, w1_ref[:NODE_DIM, :], preferred_element_type=f32)
           + jnp.dot(agg, w1_ref[NODE_DIM:NODE_DIM + EDGE_DIM, :],
                     preferred_element_type=f32)
           + jnp.dot(onehot, gh, preferred_element_type=f32)
           + b1_ref[...])
    h = jnp.maximum(acc, 0.0)
    out_ref[...] = jnp.dot(h, w2_ref[...], preferred_element_type=f32) \
        + b2_ref[...]


def kernel(nodes, edges, edge_index, graph_globals, batch, W1, b1, W2, b2):
    recv2 = _extract_recv(edge_index)
    slabs = _edges_to_slabs(edges)
    partials = _sc_segment_sum(slabs, recv2)

    batch3 = batch.reshape(N // BLK, 1, BLK)
    grid = (N // BLK,)
    out = pl.pallas_call(
        _tc_mlp_kernel,
        grid=grid,
        in_specs=[
            pl.BlockSpec((BLK, NODE_DIM), lambda i: (i, 0)),
            pl.BlockSpec((NC, BLK, EDGE_DIM), lambda i: (0, i, 0)),
            pl.BlockSpec((1, 1, BLK), lambda i: (i, 0, 0)),
            pl.BlockSpec((G, GLOBAL_DIM), lambda i: (0, 0)),
            pl.BlockSpec((NODE_DIM + EDGE_DIM + GLOBAL_DIM, HIDDEN),
                         lambda i: (0, 0)),
            pl.BlockSpec((1, HIDDEN), lambda i: (0, 0)),
            pl.BlockSpec((HIDDEN, NODE_DIM), lambda i: (0, 0)),
            pl.BlockSpec((1, NODE_DIM), lambda i: (0, 0)),
        ],
        out_specs=pl.BlockSpec((BLK, NODE_DIM), lambda i: (i, 0)),
        out_shape=jax.ShapeDtypeStruct((N, NODE_DIM), jnp.float32),
    )(nodes, partials, batch3, graph_globals, W1, b1.reshape(1, HIDDEN),
      W2, b2.reshape(1, NODE_DIM))
    return out
